# Initial kernel scaffold; baseline (speedup 1.0000x reference)
import jax
import jax.numpy as jnp
from jax import lax
from jax.experimental import pallas as pl
from jax.experimental.pallas import tpu as pltpu
from jax.experimental.pallas import tpu_sc as plsc

_N = 10000
_E = 320000
_D = 128
_L = 3
_BN_EPS = 1e-5

_NC = 2
_NS = 16
_NW = _NC * _NS
_CHUNK = 128
_NCH = 80
_EPW = _NCH * _CHUNK
_EP = _EPW * _NW
_NPAD = 10240
_ZBLK = 128
_ZPT = _NPAD // _NS // _ZBLK
_OUT_RPT = _N // _NS

_mesh = plsc.VectorSubcoreMesh(core_axis_name="c", subcore_axis_name="s")


def _agg_kernel_body(h_hbm, src_hbm, dst_hbm, zrow_hbm, out_hbm,
                     src_v, dst_v, rows_v, zb_v, acc_sh, gsem):
    c = lax.axis_index("c")
    s = lax.axis_index("s")
    wid = s * _NC + c

    pltpu.sync_copy(zrow_hbm, zb_v)

    def _zero(j, carry):
        pltpu.sync_copy(zb_v, acc_sh.at[pl.ds(s * (_NPAD // _NS) + j * _ZBLK,
                                              _ZBLK)])
        return carry

    lax.fori_loop(0, _ZPT, _zero, 0)

    pltpu.sync_copy(src_hbm.at[wid], src_v)
    pltpu.sync_copy(dst_hbm.at[wid], dst_v)
    plsc.subcore_barrier()

    def _chunk(i, carry):
        pltpu.async_copy(h_hbm.at[src_v.at[i]], rows_v, gsem).wait()
        pltpu.sync_copy(rows_v, acc_sh.at[dst_v.at[i]], add=True)
        return carry

    lax.fori_loop(0, _NCH, _chunk, 0)
    plsc.subcore_barrier()

    pltpu.sync_copy(acc_sh.at[pl.ds(s * _OUT_RPT, _OUT_RPT)],
                    out_hbm.at[c, pl.ds(s * _OUT_RPT, _OUT_RPT)])


_agg_call = pl.kernel(
    _agg_kernel_body,
    out_type=jax.ShapeDtypeStruct((_NC, _N, _D), jnp.float32),
    mesh=_mesh,
    scratch_types=[
        pltpu.VMEM((_NCH, _CHUNK), jnp.int32),
        pltpu.VMEM((_NCH, _CHUNK), jnp.int32),
        pltpu.VMEM((_CHUNK, _D), jnp.float32),
        pltpu.VMEM((_ZBLK, _D), jnp.float32),
        pltpu.VMEM_SHARED((_NPAD, _D), jnp.float32),
        pltpu.SemaphoreType.DMA,
    ],
)


def _deg_kernel_body(dst_hbm, ones_hbm, zq_hbm, out_hbm,
                     dst_v, ones_v, zq_v, dacc_sh):
    c = lax.axis_index("c")
    s = lax.axis_index("s")
    wid = s * _NC + c

    pltpu.sync_copy(zq_hbm, zq_v)

    def _zero(j, carry):
        pltpu.sync_copy(zq_v, dacc_sh.at[pl.ds(s * (_NPAD // _NS) + j * _ZBLK,
                                               _ZBLK)])
        return carry

    lax.fori_loop(0, _ZPT, _zero, 0)

    pltpu.sync_copy(dst_hbm.at[wid], dst_v)
    pltpu.sync_copy(ones_hbm, ones_v)
    plsc.subcore_barrier()

    def _chunk(i, carry):
        pltpu.sync_copy(ones_v, dacc_sh.at[dst_v.at[i]], add=True)
        return carry

    lax.fori_loop(0, _NCH, _chunk, 0)
    plsc.subcore_barrier()

    pltpu.sync_copy(dacc_sh.at[pl.ds(s * _OUT_RPT, _OUT_RPT)],
                    out_hbm.at[c, pl.ds(s * _OUT_RPT, _OUT_RPT)])


_deg_call = pl.kernel(
    _deg_kernel_body,
    out_type=jax.ShapeDtypeStruct((_NC, _N, 16), jnp.float32),
    mesh=_mesh,
    scratch_types=[
        pltpu.VMEM((_NCH, _CHUNK), jnp.int32),
        pltpu.VMEM((_CHUNK, 16), jnp.float32),
        pltpu.VMEM((_ZBLK, 16), jnp.float32),
        pltpu.VMEM_SHARED((_NPAD, 16), jnp.float32),
    ],
)


def _bn_relu(z, g, b):
    mu = jnp.mean(z, axis=0, keepdims=True)
    var = jnp.mean((z - mu) ** 2, axis=0, keepdims=True)
    return jnp.maximum((z - mu) * jax.lax.rsqrt(var + _BN_EPS) * g + b, 0.0)


def _mlp_body(h_ref, p_ref, degp_ref, w1_ref, b1_ref, g1_ref, be1_ref,
              w2_ref, b2_ref, g2_ref, be2_ref, o_ref):
    deg = degp_ref[0, :, 0:1] + degp_ref[1, :, 0:1]
    inv_deg = 1.0 / jnp.maximum(deg, 1.0)
    z = h_ref[...] + (p_ref[0] + p_ref[1]) * inv_deg
    z = jnp.dot(z, w1_ref[...], preferred_element_type=jnp.float32) + b1_ref[...]
    z = _bn_relu(z, g1_ref[...], be1_ref[...])
    z = jnp.dot(z, w2_ref[...], preferred_element_type=jnp.float32) + b2_ref[...]
    z = _bn_relu(z, g2_ref[...], be2_ref[...])
    o_ref[...] = z


_mlp_call = pl.pallas_call(
    _mlp_body,
    out_shape=jax.ShapeDtypeStruct((_N, _D), jnp.float32),
)


def kernel(x, edge_index, W1, B1, G1, Be1, W2, B2, G2, Be2):
    src = edge_index[0]
    dst = edge_index[1]
    pad = _EP - _E
    srcp = jnp.concatenate([src, jnp.zeros((pad,), jnp.int32)])
    dstp = jnp.concatenate([dst, jnp.full((pad,), _N, jnp.int32)])
    src3 = srcp.reshape(_NW, _NCH, _CHUNK)
    dst3 = dstp.reshape(_NW, _NCH, _CHUNK)
    zrow = jnp.zeros((_ZBLK, _D), jnp.float32)
    zq = jnp.zeros((_ZBLK, 16), jnp.float32)
    ones = jnp.ones((_CHUNK, 16), jnp.float32)

    degp = _deg_call(dst3, ones, zq)

    h = x
    for l in range(_L):
        p = _agg_call(h, src3, dst3, zrow)
        h = _mlp_call(h, p, degp,
                      W1[l], B1[l].reshape(1, _D), G1[l].reshape(1, _D),
                      Be1[l].reshape(1, _D),
                      W2[l], B2[l].reshape(1, _D), G2[l].reshape(1, _D),
                      Be2[l].reshape(1, _D))
    return h


# speedup vs baseline: 2.3397x; 2.3397x over previous
import jax
import jax.numpy as jnp
from jax import lax
from jax.experimental import pallas as pl
from jax.experimental.pallas import tpu as pltpu
from jax.experimental.pallas import tpu_sc as plsc

_N = 10000
_E = 320000
_D = 128
_L = 3
_BN_EPS = 1e-5

_NC = 2
_NS = 16
_NW = _NC * _NS
_CHUNK = 128
_NCH = 80
_EPW = _NCH * _CHUNK
_EP = _EPW * _NW
_NPAD = 10240
_ZBLK = 64
_ZPT = _NPAD // _NS // _ZBLK
_OUT_RPT = 632
_NOUT = _NS * _OUT_RPT

_mesh = plsc.VectorSubcoreMesh(core_axis_name="c", subcore_axis_name="s")


def _agg_kernel_body(h_hbm, src_hbm, dst_hbm, zrow_hbm, out_hbm,
                     src_v, dst_v, rows_v, zb_v, acc_sh, gsem):
    c = lax.axis_index("c")
    s = lax.axis_index("s")
    wid = s * _NC + c

    pltpu.sync_copy(zrow_hbm, zb_v)

    def _zero(j, carry):
        pltpu.sync_copy(zb_v, acc_sh.at[pl.ds(s * (_NPAD // _NS) + j * _ZBLK,
                                              _ZBLK)])
        return carry

    lax.fori_loop(0, _ZPT, _zero, 0)

    pltpu.sync_copy(src_hbm.at[wid], src_v)
    pltpu.sync_copy(dst_hbm.at[wid], dst_v)
    plsc.subcore_barrier()

    def _chunk(i, carry):
        pltpu.async_copy(h_hbm.at[src_v.at[i]], rows_v, gsem).wait()
        pltpu.sync_copy(rows_v, acc_sh.at[dst_v.at[i]], add=True)
        return carry

    lax.fori_loop(0, _NCH, _chunk, 0)
    plsc.subcore_barrier()

    pltpu.sync_copy(acc_sh.at[pl.ds(s * _OUT_RPT, _OUT_RPT)],
                    out_hbm.at[c, pl.ds(s * _OUT_RPT, _OUT_RPT)])


_agg_call = pl.kernel(
    _agg_kernel_body,
    out_type=jax.ShapeDtypeStruct((_NC, _NOUT, _D), jnp.float32),
    mesh=_mesh,
    scratch_types=[
        pltpu.VMEM((_NCH, _CHUNK), jnp.int32),
        pltpu.VMEM((_NCH, _CHUNK), jnp.int32),
        pltpu.VMEM((_CHUNK, _D), jnp.float32),
        pltpu.VMEM((_ZBLK, _D), jnp.float32),
        pltpu.VMEM_SHARED((_NPAD, _D), jnp.float32),
        pltpu.SemaphoreType.DMA,
    ],
)


def _bn_relu(z, g, b):
    mu = jnp.mean(z, axis=0, keepdims=True)
    var = jnp.mean((z - mu) ** 2, axis=0, keepdims=True)
    return jnp.maximum((z - mu) / jnp.sqrt(var + _BN_EPS) * g + b, 0.0)


def _mlp_body(h_ref, p_ref, degp_ref, w1_ref, b1_ref, g1_ref, be1_ref,
              w2_ref, b2_ref, g2_ref, be2_ref, o_ref):
    deg = degp_ref[0, :_N, 0:1] + degp_ref[1, :_N, 0:1]
    inv_deg = 1.0 / jnp.maximum(deg, 1.0)
    z = h_ref[...] + (p_ref[0, :_N] + p_ref[1, :_N]) * inv_deg
    z = jnp.dot(z, w1_ref[...], preferred_element_type=jnp.float32) + b1_ref[...]
    z = _bn_relu(z, g1_ref[...], be1_ref[...])
    z = jnp.dot(z, w2_ref[...], preferred_element_type=jnp.float32) + b2_ref[...]
    z = _bn_relu(z, g2_ref[...], be2_ref[...])
    o_ref[...] = z


_mlp_call = pl.pallas_call(
    _mlp_body,
    out_shape=jax.ShapeDtypeStruct((_N, _D), jnp.float32),
)


def kernel(x, edge_index, W1, B1, G1, Be1, W2, B2, G2, Be2):
    src = edge_index[0]
    dst = edge_index[1]
    pad = _EP - _E
    srcp = jnp.concatenate([src, jnp.zeros((pad,), jnp.int32)])
    dstp = jnp.concatenate([dst, jnp.full((pad,), _N, jnp.int32)])
    src3 = srcp.reshape(_NW, _NCH, _CHUNK)
    dst3 = dstp.reshape(_NW, _NCH, _CHUNK)
    zrow = jnp.zeros((_ZBLK, _D), jnp.float32)

    degp = _agg_call(jnp.ones((_N, _D), jnp.float32), src3, dst3, zrow)

    h = x
    for l in range(_L):
        p = _agg_call(h, src3, dst3, zrow)
        h = _mlp_call(h, p, degp,
                      W1[l], B1[l].reshape(1, _D), G1[l].reshape(1, _D),
                      Be1[l].reshape(1, _D),
                      W2[l], B2[l].reshape(1, _D), G2[l].reshape(1, _D),
                      Be2[l].reshape(1, _D))
    return h


# speedup vs baseline: 2.9418x; 1.2574x over previous
import jax
import jax.numpy as jnp
from jax import lax
from jax.experimental import pallas as pl
from jax.experimental.pallas import tpu as pltpu
from jax.experimental.pallas import tpu_sc as plsc

_N = 10000
_E = 320000
_D = 128
_L = 3
_BN_EPS = 1e-5

_NC = 2
_NS = 16
_NW = _NC * _NS
_CHUNK = 128
_NCH = 80
_EPW = _NCH * _CHUNK
_EP = _EPW * _NW
_NPAD = 10240
_ZBLK = 64
_ZPT = _NPAD // _NS // _ZBLK
_OUT_RPT = 632
_NOUT = _NS * _OUT_RPT

_DW = 128

_mesh = plsc.VectorSubcoreMesh(core_axis_name="c", subcore_axis_name="s")


def _deg_kernel_body(dst_hbm, ones_hbm, zrow_hbm, out_hbm,
                     dst_v, ones_v, zb_v, acc_sh):
    c = lax.axis_index("c")
    s = lax.axis_index("s")
    wid = s * _NC + c

    pltpu.sync_copy(zrow_hbm, zb_v)

    def _zero(j, carry):
        pltpu.sync_copy(zb_v, acc_sh.at[pl.ds(s * (_NPAD // _NS) + j * _ZBLK,
                                              _ZBLK)])
        return carry

    lax.fori_loop(0, _ZPT, _zero, 0)

    pltpu.sync_copy(dst_hbm.at[wid], dst_v)
    pltpu.sync_copy(ones_hbm, ones_v)
    plsc.subcore_barrier()

    def _chunk(i, carry):
        pltpu.sync_copy(ones_v, acc_sh.at[dst_v.at[i]], add=True)
        return carry

    lax.fori_loop(0, _NCH, _chunk, 0)
    plsc.subcore_barrier()

    pltpu.sync_copy(acc_sh.at[pl.ds(s * _OUT_RPT, _OUT_RPT)],
                    out_hbm.at[c, pl.ds(s * _OUT_RPT, _OUT_RPT)])


_deg_call = pl.kernel(
    _deg_kernel_body,
    out_type=jax.ShapeDtypeStruct((_NC, _NOUT, _DW), jnp.float32),
    mesh=_mesh,
    scratch_types=[
        pltpu.VMEM((_NCH, _CHUNK), jnp.int32),
        pltpu.VMEM((_CHUNK, _DW), jnp.float32),
        pltpu.VMEM((_ZBLK, _DW), jnp.float32),
        pltpu.VMEM_SHARED((_NPAD, _DW), jnp.float32),
    ],
)


def _agg_kernel_body(h_hbm, src_hbm, dst_hbm, zrow_hbm, out_hbm,
                     src_v, dst_v, rows_v, zb_v, acc_sh, gsem):
    c = lax.axis_index("c")
    s = lax.axis_index("s")
    wid = s * _NC + c

    pltpu.sync_copy(zrow_hbm, zb_v)

    def _zero(j, carry):
        pltpu.sync_copy(zb_v, acc_sh.at[pl.ds(s * (_NPAD // _NS) + j * _ZBLK,
                                              _ZBLK)])
        return carry

    lax.fori_loop(0, _ZPT, _zero, 0)

    pltpu.sync_copy(src_hbm.at[wid], src_v)
    pltpu.sync_copy(dst_hbm.at[wid], dst_v)
    plsc.subcore_barrier()

    def _chunk(i, carry):
        pltpu.async_copy(h_hbm.at[src_v.at[i]], rows_v, gsem).wait()
        pltpu.sync_copy(rows_v, acc_sh.at[dst_v.at[i]], add=True)
        return carry

    lax.fori_loop(0, _NCH, _chunk, 0)
    plsc.subcore_barrier()

    pltpu.sync_copy(acc_sh.at[pl.ds(s * _OUT_RPT, _OUT_RPT)],
                    out_hbm.at[c, pl.ds(s * _OUT_RPT, _OUT_RPT)])


_agg_call = pl.kernel(
    _agg_kernel_body,
    out_type=jax.ShapeDtypeStruct((_NC, _NOUT, _D), jnp.float32),
    mesh=_mesh,
    scratch_types=[
        pltpu.VMEM((_NCH, _CHUNK), jnp.int32),
        pltpu.VMEM((_NCH, _CHUNK), jnp.int32),
        pltpu.VMEM((_CHUNK, _D), jnp.float32),
        pltpu.VMEM((_ZBLK, _D), jnp.float32),
        pltpu.VMEM_SHARED((_NPAD, _D), jnp.float32),
        pltpu.SemaphoreType.DMA,
    ],
)


def _bn_relu(z, g, b):
    mu = jnp.mean(z, axis=0, keepdims=True)
    var = jnp.mean((z - mu) ** 2, axis=0, keepdims=True)
    return jnp.maximum((z - mu) / jnp.sqrt(var + _BN_EPS) * g + b, 0.0)


def _mlp_body(h_ref, p_ref, degp_ref, w1_ref, b1_ref, g1_ref, be1_ref,
              w2_ref, b2_ref, g2_ref, be2_ref, o_ref):
    deg = degp_ref[0, :_N, 0:1] + degp_ref[1, :_N, 0:1]
    inv_deg = 1.0 / jnp.maximum(deg, 1.0)
    z = h_ref[...] + (p_ref[0, :_N] + p_ref[1, :_N]) * inv_deg
    z = jnp.dot(z, w1_ref[...], preferred_element_type=jnp.float32) + b1_ref[...]
    z = _bn_relu(z, g1_ref[...], be1_ref[...])
    z = jnp.dot(z, w2_ref[...], preferred_element_type=jnp.float32) + b2_ref[...]
    z = _bn_relu(z, g2_ref[...], be2_ref[...])
    o_ref[...] = z


_mlp_call = pl.pallas_call(
    _mlp_body,
    out_shape=jax.ShapeDtypeStruct((_N, _D), jnp.float32),
)


def kernel(x, edge_index, W1, B1, G1, Be1, W2, B2, G2, Be2):
    src = edge_index[0]
    dst = edge_index[1]
    pad = _EP - _E
    srcp = jnp.concatenate([src, jnp.zeros((pad,), jnp.int32)])
    dstp = jnp.concatenate([dst, jnp.full((pad,), _N, jnp.int32)])
    src3 = srcp.reshape(_NW, _NCH, _CHUNK)
    dst3 = dstp.reshape(_NW, _NCH, _CHUNK)
    zrow = jnp.zeros((_ZBLK, _D), jnp.float32)

    degp = _deg_call(dst3, jnp.ones((_CHUNK, _DW), jnp.float32),
                     jnp.zeros((_ZBLK, _DW), jnp.float32))

    h = x
    for l in range(_L):
        p = _agg_call(h, src3, dst3, zrow)
        h = _mlp_call(h, p, degp,
                      W1[l], B1[l].reshape(1, _D), G1[l].reshape(1, _D),
                      Be1[l].reshape(1, _D),
                      W2[l], B2[l].reshape(1, _D), G2[l].reshape(1, _D),
                      Be2[l].reshape(1, _D))
    return h


# speedup vs baseline: 3.4800x; 1.1829x over previous
import jax
import jax.numpy as jnp
from jax import lax
from jax.experimental import pallas as pl
from jax.experimental.pallas import tpu as pltpu
from jax.experimental.pallas import tpu_sc as plsc

_N = 10000
_E = 320000
_D = 128
_L = 3
_BN_EPS = 1e-5

_NC = 2
_NS = 16
_NW = _NC * _NS
_CHUNK = 64
_NCH = 160
_NCHR = 80
_EPW = _NCH * _CHUNK
_EP = _EPW * _NW
_NPAD = 10240
_ZBLK = 64
_ZPT = _NPAD // _NS // _ZBLK
_OUT_RPT = 632
_NOUT = _NS * _OUT_RPT

_DW = 128

_mesh = plsc.VectorSubcoreMesh(core_axis_name="c", subcore_axis_name="s")


def _deg_kernel_body(dst_hbm, ones_hbm, zrow_hbm, out_hbm,
                     dst_v, ones_v, zb_v, acc_sh):
    c = lax.axis_index("c")
    s = lax.axis_index("s")
    wid = s * _NC + c

    pltpu.sync_copy(zrow_hbm, zb_v)

    def _zero(j, carry):
        pltpu.sync_copy(zb_v, acc_sh.at[pl.ds(s * (_NPAD // _NS) + j * _ZBLK,
                                              _ZBLK)])
        return carry

    lax.fori_loop(0, _ZPT, _zero, 0)

    pltpu.sync_copy(dst_hbm.at[wid], dst_v)
    pltpu.sync_copy(ones_hbm, ones_v)
    plsc.subcore_barrier()

    def _chunk(i, carry):
        pltpu.sync_copy(ones_v, acc_sh.at[dst_v.at[i]], add=True)
        return carry

    lax.fori_loop(0, _NCH, _chunk, 0)
    plsc.subcore_barrier()

    pltpu.sync_copy(acc_sh.at[pl.ds(s * _OUT_RPT, _OUT_RPT)],
                    out_hbm.at[c, pl.ds(s * _OUT_RPT, _OUT_RPT)])


_deg_call = pl.kernel(
    _deg_kernel_body,
    out_type=jax.ShapeDtypeStruct((_NC, _NOUT, _DW), jnp.float32),
    mesh=_mesh,
    scratch_types=[
        pltpu.VMEM((_NCH, _CHUNK), jnp.int32),
        pltpu.VMEM((_CHUNK, _DW), jnp.float32),
        pltpu.VMEM((_ZBLK, _DW), jnp.float32),
        pltpu.VMEM_SHARED((_NPAD, _DW), jnp.float32),
    ],
)


_NBUF = 2


def _agg_kernel_body(h_hbm, src_hbm, dst_hbm, zrow_hbm, out_hbm,
                     src_v, dst_v, r0, r1, acc_sh,
                     g0, g1):
    c = lax.axis_index("c")
    s = lax.axis_index("s")
    wid = s * _NC + c
    rows = (r0, r1)
    sems = (g0, g1)

    pltpu.sync_copy(zrow_hbm, r0)

    def _zero(j, carry):
        pltpu.sync_copy(r0, acc_sh.at[pl.ds(s * (_NPAD // _NS) + j * _ZBLK,
                                            _ZBLK)])
        return carry

    lax.fori_loop(0, _ZPT, _zero, 0)

    pltpu.sync_copy(src_hbm.at[wid], src_v)
    pltpu.sync_copy(dst_hbm.at[wid], dst_v)
    for b in range(_NBUF):
        pltpu.async_copy(h_hbm.at[src_v.at[0, pl.ds(b * _CHUNK, _CHUNK)]],
                         rows[b], sems[b])
    plsc.subcore_barrier()

    def _chunk(j, carry):
        for b in range(_NBUF):
            sl = pl.ds(b * _CHUNK, _CHUNK)
            pltpu.make_async_copy(h_hbm.at[src_v.at[j, sl]], rows[b],
                                  sems[b]).wait()
            pltpu.sync_copy(rows[b], acc_sh.at[dst_v.at[j, sl]], add=True)

            @pl.when(j + 1 < _NCHR)
            def _():
                pltpu.async_copy(h_hbm.at[src_v.at[j + 1, sl]], rows[b],
                                 sems[b])
        return carry

    lax.fori_loop(0, _NCHR, _chunk, 0)
    plsc.subcore_barrier()

    pltpu.sync_copy(acc_sh.at[pl.ds(s * _OUT_RPT, _OUT_RPT)],
                    out_hbm.at[c, pl.ds(s * _OUT_RPT, _OUT_RPT)])


_agg_call = pl.kernel(
    _agg_kernel_body,
    out_type=jax.ShapeDtypeStruct((_NC, _NOUT, _D), jnp.float32),
    mesh=_mesh,
    scratch_types=[
        pltpu.VMEM((_NCHR, _NBUF * _CHUNK), jnp.int32),
        pltpu.VMEM((_NCHR, _NBUF * _CHUNK), jnp.int32),
        pltpu.VMEM((_CHUNK, _D), jnp.float32),
        pltpu.VMEM((_CHUNK, _D), jnp.float32),
        pltpu.VMEM_SHARED((_NPAD, _D), jnp.float32),
        pltpu.SemaphoreType.DMA,
        pltpu.SemaphoreType.DMA,
    ],
)


def _bn_relu(z, g, b):
    mu = jnp.mean(z, axis=0, keepdims=True)
    var = jnp.mean((z - mu) ** 2, axis=0, keepdims=True)
    return jnp.maximum((z - mu) / jnp.sqrt(var + _BN_EPS) * g + b, 0.0)


def _mlp_body(h_ref, p_ref, degp_ref, w1_ref, b1_ref, g1_ref, be1_ref,
              w2_ref, b2_ref, g2_ref, be2_ref, o_ref):
    deg = degp_ref[0, :_N, 0:1] + degp_ref[1, :_N, 0:1]
    inv_deg = 1.0 / jnp.maximum(deg, 1.0)
    z = h_ref[...] + (p_ref[0, :_N] + p_ref[1, :_N]) * inv_deg
    z = jnp.dot(z, w1_ref[...], preferred_element_type=jnp.float32) + b1_ref[...]
    z = _bn_relu(z, g1_ref[...], be1_ref[...])
    z = jnp.dot(z, w2_ref[...], preferred_element_type=jnp.float32) + b2_ref[...]
    z = _bn_relu(z, g2_ref[...], be2_ref[...])
    o_ref[...] = z


_mlp_call = pl.pallas_call(
    _mlp_body,
    out_shape=jax.ShapeDtypeStruct((_N, _D), jnp.float32),
)


def kernel(x, edge_index, W1, B1, G1, Be1, W2, B2, G2, Be2):
    src = edge_index[0]
    dst = edge_index[1]
    pad = _EP - _E
    srcp = jnp.concatenate([src, jnp.zeros((pad,), jnp.int32)])
    dstp = jnp.concatenate([dst, jnp.full((pad,), _N, jnp.int32)])
    src3 = srcp.reshape(_NW, _NCHR, _NBUF * _CHUNK)
    dst3 = dstp.reshape(_NW, _NCHR, _NBUF * _CHUNK)
    dst3d = dstp.reshape(_NW, _NCH, _CHUNK)
    zrow = jnp.zeros((_ZBLK, _D), jnp.float32)

    degp = _deg_call(dst3d, jnp.ones((_CHUNK, _DW), jnp.float32),
                     jnp.zeros((_ZBLK, _DW), jnp.float32))

    h = x
    for l in range(_L):
        p = _agg_call(h, src3, dst3, zrow)
        h = _mlp_call(h, p, degp,
                      W1[l], B1[l].reshape(1, _D), G1[l].reshape(1, _D),
                      Be1[l].reshape(1, _D),
                      W2[l], B2[l].reshape(1, _D), G2[l].reshape(1, _D),
                      Be2[l].reshape(1, _D))
    return h
